# baseline (speedup 1.0000x reference)
import functools

import jax
import jax.numpy as jnp
from jax import lax
from jax.experimental import pallas as pl
from jax.experimental.pallas import tpu as pltpu
from jax.experimental.pallas import tpu_sc as plsc

VOCAB = 100000
HIDDEN = 768
MAX_POS = 8192
BATCH = 4
SEQ = 4096
EPS = 1e-5

L = 16
NC, NS = 2, 16
NW = NC * NS
NTOK = BATCH * SEQ
TOK_PER_W = NTOK // NW
T = 64
NCHUNK = TOK_PER_W // T
NCH = HIDDEN // L


_GDN = lax.GatherDimensionNumbers(
    offset_dims=(), collapsed_slice_dims=(0,), start_index_map=(0,))


def _lane_sum(v):
    for sh in (8, 4, 2, 1):
        idx = (jnp.arange(L, dtype=jnp.int32) + sh) % L
        perm = lax.gather(v, idx[:, None], _GDN, (1,),
                          mode=lax.GatherScatterMode.PROMISE_IN_BOUNDS)
        v = v + perm
    return v


def _rsqrt_vec(v):
    bits = lax.bitcast_convert_type(v, jnp.int32)
    y = lax.bitcast_convert_type(jnp.int32(0x5F3759DF) - (bits >> 1),
                                 jnp.float32)
    for _ in range(3):
        y = y * (1.5 - 0.5 * v * y * y)
    return y


@functools.partial(
    pl.kernel,
    mesh=plsc.VectorSubcoreMesh(core_axis_name="c", subcore_axis_name="s"),
    out_type=jax.ShapeDtypeStruct((NTOK, HIDDEN), jnp.float32),
    scratch_types=[
        pltpu.VMEM((T,), jnp.int32),
        pltpu.VMEM((T, HIDDEN), jnp.float32),
        pltpu.VMEM((T, HIDDEN), jnp.float32),
        pltpu.VMEM((HIDDEN,), jnp.float32),
        pltpu.VMEM((HIDDEN,), jnp.float32),
        pltpu.SemaphoreType.DMA,
    ],
)
def _embed_ln_kernel(x_hbm, tok_tbl, pos_tbl, gam_hbm, bet_hbm, out_hbm,
                     idx_v, rows_v, pos_v, gam_v, bet_v, sem):
    wid = lax.axis_index("s") * NC + lax.axis_index("c")
    pltpu.sync_copy(gam_hbm, gam_v)
    pltpu.sync_copy(bet_hbm, bet_v)

    def chunk_body(ch, carry):
        base = wid * TOK_PER_W + ch * T
        pos_base = lax.rem(base, SEQ)
        pltpu.sync_copy(x_hbm.at[pl.ds(base, T)], idx_v)
        pltpu.async_copy(tok_tbl.at[idx_v], rows_v, sem).wait()
        pltpu.sync_copy(pos_tbl.at[pl.ds(pos_base, T), :], pos_v)

        def tok_body(t, c2):
            acc_s = jnp.zeros((L,), jnp.float32)
            acc_q = jnp.zeros((L,), jnp.float32)
            for c in range(NCH):
                v = rows_v[t, pl.ds(c * L, L)] + pos_v[t, pl.ds(c * L, L)]
                rows_v[t, pl.ds(c * L, L)] = v
                acc_s = acc_s + v
                acc_q = acc_q + v * v
            meanv = _lane_sum(acc_s) * (1.0 / HIDDEN)
            varv = _lane_sum(acc_q) * (1.0 / HIDDEN) - meanv * meanv
            rstd = _rsqrt_vec(varv + EPS)
            for c in range(NCH):
                v = rows_v[t, pl.ds(c * L, L)]
                y = ((v - meanv) * rstd) * gam_v[pl.ds(c * L, L)] \
                    + bet_v[pl.ds(c * L, L)]
                rows_v[t, pl.ds(c * L, L)] = y
            return c2

        lax.fori_loop(0, T, tok_body, 0)
        pltpu.sync_copy(rows_v, out_hbm.at[pl.ds(base, T), :])
        return carry

    lax.fori_loop(0, NCHUNK, chunk_body, 0)


def kernel(x, token_table, pos_table, ln_gamma, ln_beta):
    x_flat = x.reshape(-1).astype(jnp.int32)
    out = _embed_ln_kernel(x_flat, token_table, pos_table, ln_gamma, ln_beta)
    return out.reshape(BATCH, SEQ, HIDDEN)


# speedup vs baseline: 1.2124x; 1.2124x over previous
import functools

import jax
import jax.numpy as jnp
from jax import lax
from jax.experimental import pallas as pl
from jax.experimental.pallas import tpu as pltpu
from jax.experimental.pallas import tpu_sc as plsc

VOCAB = 100000
HIDDEN = 768
MAX_POS = 8192
BATCH = 4
SEQ = 4096
EPS = 1e-5

L = 16
NC, NS = 2, 16
NW = NC * NS
NTOK = BATCH * SEQ
TOK_PER_W = NTOK // NW
T = 64
NCHUNK = TOK_PER_W // T
NCH = HIDDEN // L
NGRP = T // L
CG = 8
NCG = NCH // CG

_GDN = lax.GatherDimensionNumbers(
    offset_dims=(), collapsed_slice_dims=(0,), start_index_map=(0,))


def _lane_sum(v):
    for sh in (8, 4, 2, 1):
        idx = (jnp.arange(L, dtype=jnp.int32) + sh) % L
        perm = lax.gather(v, idx[:, None], _GDN, (1,),
                          mode=lax.GatherScatterMode.PROMISE_IN_BOUNDS)
        v = v + perm
    return v


def _rsqrt_vec(v):
    bits = lax.bitcast_convert_type(v, jnp.int32)
    y = lax.bitcast_convert_type(jnp.int32(0x5F3759DF) - (bits >> 1),
                                 jnp.float32)
    for _ in range(3):
        y = y * (1.5 - 0.5 * v * y * y)
    return y


@functools.partial(
    pl.kernel,
    mesh=plsc.VectorSubcoreMesh(core_axis_name="c", subcore_axis_name="s"),
    out_type=jax.ShapeDtypeStruct((NTOK, HIDDEN), jnp.float32),
    scratch_types=[
        pltpu.VMEM((T,), jnp.int32),
        pltpu.VMEM((T, HIDDEN), jnp.float32),
        pltpu.VMEM((T, HIDDEN), jnp.float32),
        pltpu.VMEM((HIDDEN,), jnp.float32),
        pltpu.VMEM((HIDDEN,), jnp.float32),
        pltpu.VMEM((T, L), jnp.float32),
        pltpu.VMEM((T, L), jnp.float32),
        pltpu.SemaphoreType.DMA,
    ],
)
def _embed_ln_kernel(x_hbm, tok_tbl, pos_tbl, gam_hbm, bet_hbm, out_hbm,
                     idx_v, rows_v, pos_v, gam_v, bet_v,
                     p_v, q_v, sem):
    wid = lax.axis_index("s") * NC + lax.axis_index("c")
    pltpu.sync_copy(gam_hbm, gam_v)
    pltpu.sync_copy(bet_hbm, bet_v)

    def chunk_body(ch, carry):
        base = wid * TOK_PER_W + ch * T
        pos_base = lax.rem(base, SEQ)
        pltpu.sync_copy(x_hbm.at[pl.ds(base, T)], idx_v)
        pltpu.async_copy(tok_tbl.at[idx_v], rows_v, sem).wait()
        pltpu.sync_copy(pos_tbl.at[pl.ds(pos_base, T), :], pos_v)

        @plsc.parallel_loop(0, T)
        def phase_a(t):
            accs = [jnp.zeros((L,), jnp.float32) for _ in range(4)]
            accq = [jnp.zeros((L,), jnp.float32) for _ in range(4)]
            for c in range(NCH):
                v = rows_v[t, pl.ds(c * L, L)] + pos_v[t, pl.ds(c * L, L)]
                rows_v[t, pl.ds(c * L, L)] = v
                accs[c % 4] = accs[c % 4] + v
                accq[c % 4] = accq[c % 4] + v * v
            acc_s = (accs[0] + accs[1]) + (accs[2] + accs[3])
            acc_q = (accq[0] + accq[1]) + (accq[2] + accq[3])
            meanv = _lane_sum(acc_s) * (1.0 / HIDDEN)
            varv = _lane_sum(acc_q) * (1.0 / HIDDEN) - meanv * meanv
            rstdv = _rsqrt_vec(varv + EPS)
            p_v[t, :] = rstdv
            q_v[t, :] = meanv * rstdv

        for cg in range(NCG):
            gs = [gam_v[pl.ds((cg * CG + j) * L, L)] for j in range(CG)]
            bs = [bet_v[pl.ds((cg * CG + j) * L, L)] for j in range(CG)]

            @plsc.parallel_loop(0, T)
            def phase_b(t):
                p = p_v[t, :]
                q = q_v[t, :]
                for j in range(CG):
                    c = cg * CG + j
                    v = rows_v[t, pl.ds(c * L, L)]
                    rows_v[t, pl.ds(c * L, L)] = (v * p - q) * gs[j] + bs[j]

        pltpu.sync_copy(rows_v, out_hbm.at[pl.ds(base, T), :])
        return carry

    lax.fori_loop(0, NCHUNK, chunk_body, 0)


def kernel(x, token_table, pos_table, ln_gamma, ln_beta):
    x_flat = x.reshape(-1).astype(jnp.int32)
    out = _embed_ln_kernel(x_flat, token_table, pos_table, ln_gamma, ln_beta)
    return out.reshape(BATCH, SEQ, HIDDEN)


# speedup vs baseline: 1.6360x; 1.3495x over previous
import functools

import jax
import jax.numpy as jnp
from jax import lax
from jax.experimental import pallas as pl
from jax.experimental.pallas import tpu as pltpu
from jax.experimental.pallas import tpu_sc as plsc

VOCAB = 100000
HIDDEN = 768
MAX_POS = 8192
BATCH = 4
SEQ = 4096
EPS = 1e-5

L = 16
NC, NS = 2, 16
NW = NC * NS
NTOK = BATCH * SEQ
TOK_PER_W = NTOK // NW
T = 64
NCHUNK = TOK_PER_W // T
NCH = HIDDEN // L
NGRP = T // L
CG = 8
NCG = NCH // CG

_GDN = lax.GatherDimensionNumbers(
    offset_dims=(), collapsed_slice_dims=(0,), start_index_map=(0,))


def _lane_sum(v):
    for sh in (8, 4, 2, 1):
        idx = (jnp.arange(L, dtype=jnp.int32) + sh) % L
        perm = lax.gather(v, idx[:, None], _GDN, (1,),
                          mode=lax.GatherScatterMode.PROMISE_IN_BOUNDS)
        v = v + perm
    return v


def _rsqrt_vec(v):
    bits = lax.bitcast_convert_type(v, jnp.int32)
    y = lax.bitcast_convert_type(jnp.int32(0x5F3759DF) - (bits >> 1),
                                 jnp.float32)
    for _ in range(3):
        y = y * (1.5 - 0.5 * v * y * y)
    return y


@functools.partial(
    pl.kernel,
    mesh=plsc.VectorSubcoreMesh(core_axis_name="c", subcore_axis_name="s"),
    out_type=jax.ShapeDtypeStruct((NTOK, HIDDEN), jnp.float32),
    scratch_types=[
        pltpu.VMEM((T,), jnp.int32),
        pltpu.VMEM((T, HIDDEN), jnp.float32),
        pltpu.VMEM((T, HIDDEN), jnp.float32),
        pltpu.VMEM((HIDDEN,), jnp.float32),
        pltpu.VMEM((HIDDEN,), jnp.float32),
        pltpu.VMEM((T * L,), jnp.float32),
        pltpu.VMEM((T * L,), jnp.float32),
        pltpu.VMEM((T * L,), jnp.float32),
        pltpu.VMEM((T * L,), jnp.float32),
        pltpu.SemaphoreType.DMA,
    ],
)
def _embed_ln_kernel(x_hbm, tok_tbl, pos_tbl, gam_hbm, bet_hbm, out_hbm,
                     idx_v, rows_v, pos_v, gam_v, bet_v,
                     sum_v, sq_v, p_v, q_v, sem):
    wid = lax.axis_index("s") * NC + lax.axis_index("c")
    pltpu.sync_copy(gam_hbm, gam_v)
    pltpu.sync_copy(bet_hbm, bet_v)

    def chunk_body(ch, carry):
        base = wid * TOK_PER_W + ch * T
        pos_base = lax.rem(base, SEQ)
        pltpu.sync_copy(x_hbm.at[pl.ds(base, T)], idx_v)
        pltpu.async_copy(tok_tbl.at[idx_v], rows_v, sem).wait()
        pltpu.sync_copy(pos_tbl.at[pl.ds(pos_base, T), :], pos_v)

        @plsc.parallel_loop(0, T)
        def phase_a(t):
            accs = [jnp.zeros((L,), jnp.float32) for _ in range(4)]
            accq = [jnp.zeros((L,), jnp.float32) for _ in range(4)]
            for c in range(NCH):
                v = rows_v[t, pl.ds(c * L, L)] + pos_v[t, pl.ds(c * L, L)]
                rows_v[t, pl.ds(c * L, L)] = v
                accs[c % 4] = accs[c % 4] + v
                accq[c % 4] = accq[c % 4] + v * v
            sum_v[pl.ds(t * L, L)] = (accs[0] + accs[1]) + (accs[2] + accs[3])
            sq_v[pl.ds(t * L, L)] = (accq[0] + accq[1]) + (accq[2] + accq[3])

        @plsc.parallel_loop(0, T)
        def phase_s(t):
            meanv = _lane_sum(sum_v[pl.ds(t * L, L)]) * (1.0 / HIDDEN)
            varv = _lane_sum(sq_v[pl.ds(t * L, L)]) * (1.0 / HIDDEN) - meanv * meanv
            rstdv = _rsqrt_vec(varv + EPS)
            p_v[pl.ds(t * L, L)] = rstdv
            q_v[pl.ds(t * L, L)] = meanv * rstdv

        for cg in range(NCG):
            gs = [gam_v[pl.ds((cg * CG + j) * L, L)] for j in range(CG)]
            bs = [bet_v[pl.ds((cg * CG + j) * L, L)] for j in range(CG)]

            @plsc.parallel_loop(0, T)
            def phase_b(t):
                p = p_v[pl.ds(t * L, L)]
                q = q_v[pl.ds(t * L, L)]
                for j in range(CG):
                    c = cg * CG + j
                    v = rows_v[t, pl.ds(c * L, L)]
                    rows_v[t, pl.ds(c * L, L)] = (v * p - q) * gs[j] + bs[j]

        pltpu.sync_copy(rows_v, out_hbm.at[pl.ds(base, T), :])
        return carry

    lax.fori_loop(0, NCHUNK, chunk_body, 0)


def kernel(x, token_table, pos_table, ln_gamma, ln_beta):
    x_flat = x.reshape(-1).astype(jnp.int32)
    out = _embed_ln_kernel(x_flat, token_table, pos_table, ln_gamma, ln_beta)
    return out.reshape(BATCH, SEQ, HIDDEN)


# speedup vs baseline: 2.0752x; 1.2684x over previous
import functools

import jax
import jax.numpy as jnp
from jax import lax
from jax.experimental import pallas as pl
from jax.experimental.pallas import tpu as pltpu
from jax.experimental.pallas import tpu_sc as plsc

VOCAB = 100000
HIDDEN = 768
MAX_POS = 8192
BATCH = 4
SEQ = 4096
EPS = 1e-5

L = 16
NC, NS = 2, 16
NW = NC * NS
NTOK = BATCH * SEQ
TOK_PER_W = NTOK // NW
T = 32
NCHUNK = TOK_PER_W // T
NCH = HIDDEN // L
CG = 8
NCG = NCH // CG

_GDN = lax.GatherDimensionNumbers(
    offset_dims=(), collapsed_slice_dims=(0,), start_index_map=(0,))


def _lane_sum(v):
    for sh in (8, 4, 2, 1):
        idx = (jnp.arange(L, dtype=jnp.int32) + sh) % L
        perm = lax.gather(v, idx[:, None], _GDN, (1,),
                          mode=lax.GatherScatterMode.PROMISE_IN_BOUNDS)
        v = v + perm
    return v


def _rsqrt_vec(v):
    bits = lax.bitcast_convert_type(v, jnp.int32)
    y = lax.bitcast_convert_type(jnp.int32(0x5F3759DF) - (bits >> 1),
                                 jnp.float32)
    for _ in range(3):
        y = y * (1.5 - 0.5 * v * y * y)
    return y


@functools.partial(
    pl.kernel,
    mesh=plsc.VectorSubcoreMesh(core_axis_name="c", subcore_axis_name="s"),
    out_type=jax.ShapeDtypeStruct((NTOK, HIDDEN), jnp.float32),
    scratch_types=[
        pltpu.VMEM((TOK_PER_W,), jnp.int32),
        pltpu.VMEM((2, T, HIDDEN), jnp.float32),
        pltpu.VMEM((T, HIDDEN), jnp.float32),
        pltpu.VMEM((T, HIDDEN), jnp.float32),
        pltpu.VMEM((HIDDEN,), jnp.float32),
        pltpu.VMEM((HIDDEN,), jnp.float32),
        pltpu.VMEM((T * L,), jnp.float32),
        pltpu.VMEM((T * L,), jnp.float32),
        pltpu.VMEM((T * L,), jnp.float32),
        pltpu.VMEM((T * L,), jnp.float32),
        pltpu.SemaphoreType.DMA,
        pltpu.SemaphoreType.DMA,
        pltpu.SemaphoreType.DMA,
        pltpu.SemaphoreType.DMA,
        pltpu.SemaphoreType.DMA,
    ],
)
def _embed_ln_kernel(x_hbm, tok_tbl, pos_tbl, gam_hbm, bet_hbm, out_hbm,
                     idx_v, rows2, pos_v, vbuf, gam_v, bet_v,
                     sum_v, sq_v, p_v, q_v,
                     semg0, semg1, semp, semw0, semw1):
    wid = lax.axis_index("s") * NC + lax.axis_index("c")
    w0 = wid * TOK_PER_W
    pltpu.sync_copy(gam_hbm, gam_v)
    pltpu.sync_copy(bet_hbm, bet_v)
    pltpu.sync_copy(x_hbm.at[pl.ds(w0, TOK_PER_W)], idx_v)
    semg = (semg0, semg1)
    semw = (semw0, semw1)

    def gather_in(ch, slot):
        pltpu.async_copy(tok_tbl.at[idx_v.at[pl.ds(ch * T, T)]],
                         rows2.at[slot], semg[slot])

    def pos_in(ch):
        pos_base = lax.rem(w0 + ch * T, SEQ)
        pltpu.async_copy(pos_tbl.at[pl.ds(pos_base, T), :], pos_v, semp)

    def wait_in(ch, slot):
        pltpu.make_async_copy(tok_tbl.at[idx_v.at[pl.ds(ch * T, T)]],
                              rows2.at[slot], semg[slot]).wait()
        pos_base = lax.rem(w0 + ch * T, SEQ)
        pltpu.make_async_copy(pos_tbl.at[pl.ds(pos_base, T), :],
                              pos_v, semp).wait()

    def writeout(ch, slot):
        pltpu.async_copy(rows2.at[slot],
                         out_hbm.at[pl.ds(w0 + ch * T, T), :], semw[slot])

    def wait_writeout(ch, slot):
        pltpu.make_async_copy(rows2.at[slot],
                              out_hbm.at[pl.ds(w0 + ch * T, T), :],
                              semw[slot]).wait()

    gather_in(0, 0)
    pos_in(0)

    def compute_chunk(ch, slot):
        rows_v = rows2.at[slot]
        wait_in(ch, slot)

        @plsc.parallel_loop(0, T)
        def phase_a(t):
            accs = [jnp.zeros((L,), jnp.float32) for _ in range(4)]
            accq = [jnp.zeros((L,), jnp.float32) for _ in range(4)]
            for c in range(NCH):
                v = rows_v[t, pl.ds(c * L, L)] + pos_v[t, pl.ds(c * L, L)]
                vbuf[t, pl.ds(c * L, L)] = v
                accs[c % 4] = accs[c % 4] + v
                accq[c % 4] = accq[c % 4] + v * v
            sum_v[pl.ds(t * L, L)] = (accs[0] + accs[1]) + (accs[2] + accs[3])
            sq_v[pl.ds(t * L, L)] = (accq[0] + accq[1]) + (accq[2] + accq[3])

        @pl.when(ch < NCHUNK - 1)
        def _():
            @pl.when(ch >= 1)
            def _():
                wait_writeout(ch - 1, 1 - slot)
            gather_in(ch + 1, 1 - slot)
            pos_in(ch + 1)

        @plsc.parallel_loop(0, T)
        def phase_s(t):
            meanv = _lane_sum(sum_v[pl.ds(t * L, L)]) * (1.0 / HIDDEN)
            varv = (_lane_sum(sq_v[pl.ds(t * L, L)]) * (1.0 / HIDDEN)
                    - meanv * meanv)
            rstdv = _rsqrt_vec(varv + EPS)
            p_v[pl.ds(t * L, L)] = rstdv
            q_v[pl.ds(t * L, L)] = meanv * rstdv

        for cg in range(NCG):
            gs = [gam_v[pl.ds((cg * CG + j) * L, L)] for j in range(CG)]
            bs = [bet_v[pl.ds((cg * CG + j) * L, L)] for j in range(CG)]

            @plsc.parallel_loop(0, T)
            def phase_b(t):
                p = p_v[pl.ds(t * L, L)]
                q = q_v[pl.ds(t * L, L)]
                for j in range(CG):
                    c = cg * CG + j
                    v = vbuf[t, pl.ds(c * L, L)]
                    rows_v[t, pl.ds(c * L, L)] = (v * p - q) * gs[j] + bs[j]

        writeout(ch, slot)

    def pair_body(i, carry):
        compute_chunk(2 * i, 0)
        compute_chunk(2 * i + 1, 1)
        return carry

    lax.fori_loop(0, NCHUNK // 2, pair_body, 0)
    wait_writeout(NCHUNK - 2, 0)
    wait_writeout(NCHUNK - 1, 1)


def kernel(x, token_table, pos_table, ln_gamma, ln_beta):
    x_flat = x.reshape(-1).astype(jnp.int32)
    out = _embed_ln_kernel(x_flat, token_table, pos_table, ln_gamma, ln_beta)
    return out.reshape(BATCH, SEQ, HIDDEN)


# speedup vs baseline: 2.5591x; 1.2332x over previous
import functools

import jax
import jax.numpy as jnp
from jax import lax
from jax.experimental import pallas as pl
from jax.experimental.pallas import tpu as pltpu
from jax.experimental.pallas import tpu_sc as plsc

VOCAB = 100000
HIDDEN = 768
MAX_POS = 8192
BATCH = 4
SEQ = 4096
EPS = 1e-5

L = 16
NC, NS = 2, 16
NW = NC * NS
NTOK = BATCH * SEQ
TOK_PER_W = NTOK // NW
T = 32
NCHUNK = TOK_PER_W // T
NCH = HIDDEN // L
CG = 8
NCG = NCH // CG

_GDN = lax.GatherDimensionNumbers(
    offset_dims=(), collapsed_slice_dims=(0,), start_index_map=(0,))


def _lane_sum(v):
    for sh in (8, 4, 2, 1):
        idx = (jnp.arange(L, dtype=jnp.int32) + sh) % L
        perm = lax.gather(v, idx[:, None], _GDN, (1,),
                          mode=lax.GatherScatterMode.PROMISE_IN_BOUNDS)
        v = v + perm
    return v


def _rsqrt_vec(v):
    bits = lax.bitcast_convert_type(v, jnp.int32)
    y = lax.bitcast_convert_type(jnp.int32(0x5F3759DF) - (bits >> 1),
                                 jnp.float32)
    for _ in range(3):
        y = y * (1.5 - 0.5 * v * y * y)
    return y


@functools.partial(
    pl.kernel,
    mesh=plsc.VectorSubcoreMesh(core_axis_name="c", subcore_axis_name="s"),
    out_type=jax.ShapeDtypeStruct((NTOK, HIDDEN), jnp.float32),
    scratch_types=[
        pltpu.VMEM((TOK_PER_W,), jnp.int32),
        pltpu.VMEM((2, T, HIDDEN), jnp.float32),
        pltpu.VMEM((T, HIDDEN), jnp.float32),
        pltpu.VMEM((HIDDEN,), jnp.float32),
        pltpu.VMEM((HIDDEN,), jnp.float32),
        pltpu.VMEM((T * L,), jnp.float32),
        pltpu.VMEM((T * L,), jnp.float32),
        pltpu.VMEM((T * L,), jnp.float32),
        pltpu.VMEM((T * L,), jnp.float32),
        pltpu.VMEM((T * L,), jnp.float32),
        pltpu.VMEM((T * L,), jnp.float32),
        pltpu.SemaphoreType.DMA,
        pltpu.SemaphoreType.DMA,
        pltpu.SemaphoreType.DMA,
        pltpu.SemaphoreType.DMA,
        pltpu.SemaphoreType.DMA,
    ],
)
def _embed_ln_kernel(x_hbm, tok_tbl, pos_tbl, gam_hbm, bet_hbm, out_hbm,
                     idx_v, rows2, pos_v, gam_v, bet_v,
                     sum_v, sq_v, sum2_v, sq2_v, p_v, q_v,
                     semg0, semg1, semp, semw0, semw1):
    wid = lax.axis_index("s") * NC + lax.axis_index("c")
    w0 = wid * TOK_PER_W
    pltpu.sync_copy(gam_hbm, gam_v)
    pltpu.sync_copy(bet_hbm, bet_v)
    pltpu.sync_copy(x_hbm.at[pl.ds(w0, TOK_PER_W)], idx_v)
    semg = (semg0, semg1)
    semw = (semw0, semw1)

    def gather_in(ch, slot):
        pltpu.async_copy(tok_tbl.at[idx_v.at[pl.ds(ch * T, T)]],
                         rows2.at[slot], semg[slot])

    def pos_in(ch):
        pos_base = lax.rem(w0 + ch * T, SEQ)
        pltpu.async_copy(pos_tbl.at[pl.ds(pos_base, T), :], pos_v, semp)

    def wait_in(ch, slot):
        pltpu.make_async_copy(tok_tbl.at[idx_v.at[pl.ds(ch * T, T)]],
                              rows2.at[slot], semg[slot]).wait()
        pos_base = lax.rem(w0 + ch * T, SEQ)
        pltpu.make_async_copy(pos_tbl.at[pl.ds(pos_base, T), :],
                              pos_v, semp).wait()

    def writeout(ch, slot):
        pltpu.async_copy(rows2.at[slot],
                         out_hbm.at[pl.ds(w0 + ch * T, T), :], semw[slot])

    def wait_writeout(ch, slot):
        pltpu.make_async_copy(rows2.at[slot],
                              out_hbm.at[pl.ds(w0 + ch * T, T), :],
                              semw[slot]).wait()

    gather_in(0, 0)
    pos_in(0)

    def compute_chunk(ch, slot):
        rows_v = rows2.at[slot]
        wait_in(ch, slot)

        def make_half(c0, s_ref, q_ref):
            @plsc.parallel_loop(0, T)
            def half(t):
                accs = [jnp.zeros((L,), jnp.float32) for _ in range(2)]
                accq = [jnp.zeros((L,), jnp.float32) for _ in range(2)]
                for c in range(c0, c0 + NCH // 2):
                    v = rows_v[t, pl.ds(c * L, L)] + pos_v[t, pl.ds(c * L, L)]
                    rows_v[t, pl.ds(c * L, L)] = v
                    accs[c % 2] = accs[c % 2] + v
                    accq[c % 2] = accq[c % 2] + v * v
                s_ref[pl.ds(t * L, L)] = accs[0] + accs[1]
                q_ref[pl.ds(t * L, L)] = accq[0] + accq[1]

        make_half(0, sum_v, sq_v)
        make_half(NCH // 2, sum2_v, sq2_v)

        @pl.when(ch < NCHUNK - 1)
        def _():
            @pl.when(ch >= 1)
            def _():
                wait_writeout(ch - 1, 1 - slot)
            gather_in(ch + 1, 1 - slot)
            pos_in(ch + 1)

        @plsc.parallel_loop(0, T)
        def phase_s(t):
            s_tot = sum_v[pl.ds(t * L, L)] + sum2_v[pl.ds(t * L, L)]
            q_tot = sq_v[pl.ds(t * L, L)] + sq2_v[pl.ds(t * L, L)]
            meanv = _lane_sum(s_tot) * (1.0 / HIDDEN)
            varv = _lane_sum(q_tot) * (1.0 / HIDDEN) - meanv * meanv
            rstdv = _rsqrt_vec(varv + EPS)
            p_v[pl.ds(t * L, L)] = rstdv
            q_v[pl.ds(t * L, L)] = meanv * rstdv

        for cg in range(NCG):
            gs = [gam_v[pl.ds((cg * CG + j) * L, L)] for j in range(CG)]
            bs = [bet_v[pl.ds((cg * CG + j) * L, L)] for j in range(CG)]

            @plsc.parallel_loop(0, T)
            def phase_b(t):
                p = p_v[pl.ds(t * L, L)]
                q = q_v[pl.ds(t * L, L)]
                for j in range(CG):
                    c = cg * CG + j
                    v = rows_v[t, pl.ds(c * L, L)]
                    rows_v[t, pl.ds(c * L, L)] = (v * p - q) * gs[j] + bs[j]

        writeout(ch, slot)

    def pair_body(i, carry):
        compute_chunk(2 * i, 0)
        compute_chunk(2 * i + 1, 1)
        return carry

    lax.fori_loop(0, NCHUNK // 2, pair_body, 0)
    wait_writeout(NCHUNK - 2, 0)
    wait_writeout(NCHUNK - 1, 1)


def kernel(x, token_table, pos_table, ln_gamma, ln_beta):
    x_flat = x.reshape(-1).astype(jnp.int32)
    out = _embed_ln_kernel(x_flat, token_table, pos_table, ln_gamma, ln_beta)
    return out.reshape(BATCH, SEQ, HIDDEN)


# speedup vs baseline: 2.7484x; 1.0740x over previous
import functools

import jax
import jax.numpy as jnp
from jax import lax
from jax.experimental import pallas as pl
from jax.experimental.pallas import tpu as pltpu
from jax.experimental.pallas import tpu_sc as plsc

VOCAB = 100000
HIDDEN = 768
MAX_POS = 8192
BATCH = 4
SEQ = 4096
EPS = 1e-5

L = 16
NC, NS = 2, 16
NW = NC * NS
NTOK = BATCH * SEQ
TOK_PER_W = NTOK // NW
T = 32
NCHUNK = TOK_PER_W // T
NCH = HIDDEN // L
CG = 8
NCG = NCH // CG

_GDN = lax.GatherDimensionNumbers(
    offset_dims=(), collapsed_slice_dims=(0,), start_index_map=(0,))


def _lane_sum(v):
    for sh in (8, 4, 2, 1):
        idx = (jnp.arange(L, dtype=jnp.int32) + sh) % L
        perm = lax.gather(v, idx[:, None], _GDN, (1,),
                          mode=lax.GatherScatterMode.PROMISE_IN_BOUNDS)
        v = v + perm
    return v


def _rsqrt_vec(v):
    bits = lax.bitcast_convert_type(v, jnp.int32)
    y = lax.bitcast_convert_type(jnp.int32(0x5F3759DF) - (bits >> 1),
                                 jnp.float32)
    for _ in range(3):
        y = y * (1.5 - 0.5 * v * y * y)
    return y


@functools.partial(
    pl.kernel,
    mesh=plsc.VectorSubcoreMesh(core_axis_name="c", subcore_axis_name="s"),
    out_type=jax.ShapeDtypeStruct((NTOK, HIDDEN), jnp.float32),
    scratch_types=[
        pltpu.VMEM((TOK_PER_W,), jnp.int32),
        pltpu.VMEM((2, T, HIDDEN), jnp.float32),
        pltpu.VMEM((T, HIDDEN), jnp.float32),
        pltpu.VMEM((HIDDEN,), jnp.float32),
        pltpu.VMEM((HIDDEN,), jnp.float32),
        pltpu.VMEM((T * L,), jnp.float32),
        pltpu.VMEM((T * L,), jnp.float32),
        pltpu.VMEM((T * L,), jnp.float32),
        pltpu.VMEM((T * L,), jnp.float32),
        pltpu.VMEM((T * L,), jnp.float32),
        pltpu.VMEM((T * L,), jnp.float32),
        pltpu.SemaphoreType.DMA,
        pltpu.SemaphoreType.DMA,
        pltpu.SemaphoreType.DMA,
        pltpu.SemaphoreType.DMA,
        pltpu.SemaphoreType.DMA,
    ],
)
def _embed_ln_kernel(x_hbm, tok_tbl, pos_tbl, gam_hbm, bet_hbm, out_hbm,
                     idx_v, rows2, pos_v, gam_v, bet_v,
                     sum_v, sq_v, sum2_v, sq2_v, p_v, q_v,
                     semg0, semg1, semp, semw0, semw1):
    wid = lax.axis_index("s") * NC + lax.axis_index("c")
    w0 = wid * (SEQ // NW)
    pltpu.sync_copy(gam_hbm, gam_v)
    pltpu.sync_copy(bet_hbm, bet_v)
    for b in range(BATCH):
        pltpu.sync_copy(x_hbm.at[pl.ds(b * SEQ + w0, SEQ // NW)],
                        idx_v.at[pl.ds(b * (SEQ // NW), SEQ // NW)])
    semg = (semg0, semg1)
    semw = (semw0, semw1)

    def idx_off(ch):
        jn = ch // BATCH
        bn = lax.rem(ch, BATCH)
        return bn * (SEQ // NW) + jn * T

    def out_base(ch):
        jn = ch // BATCH
        bn = lax.rem(ch, BATCH)
        return bn * SEQ + w0 + jn * T

    def gather_in(ch, slot):
        pltpu.async_copy(tok_tbl.at[idx_v.at[pl.ds(idx_off(ch), T)]],
                         rows2.at[slot], semg[slot])

    def wait_gather(ch, slot):
        pltpu.make_async_copy(tok_tbl.at[idx_v.at[pl.ds(idx_off(ch), T)]],
                              rows2.at[slot], semg[slot]).wait()

    def pos_in(j):
        pltpu.async_copy(pos_tbl.at[pl.ds(w0 + j * T, T), :], pos_v, semp)

    def wait_pos(j):
        pltpu.make_async_copy(pos_tbl.at[pl.ds(w0 + j * T, T), :],
                              pos_v, semp).wait()

    def writeout(ch, slot):
        pltpu.async_copy(rows2.at[slot],
                         out_hbm.at[pl.ds(out_base(ch), T), :], semw[slot])

    def wait_writeout(ch, slot):
        pltpu.make_async_copy(rows2.at[slot],
                              out_hbm.at[pl.ds(out_base(ch), T), :],
                              semw[slot]).wait()

    gather_in(0, 0)
    pos_in(0)

    def compute_chunk(j, b):
        slot = b % 2
        ch = BATCH * j + b
        rows_v = rows2.at[slot]
        wait_gather(ch, slot)
        if b == 0:
            wait_pos(j)

        def make_half(c0, s_ref, q_ref):
            @plsc.parallel_loop(0, T)
            def half(t):
                accs = [jnp.zeros((L,), jnp.float32) for _ in range(2)]
                accq = [jnp.zeros((L,), jnp.float32) for _ in range(2)]
                for c in range(c0, c0 + NCH // 2):
                    v = rows_v[t, pl.ds(c * L, L)] + pos_v[t, pl.ds(c * L, L)]
                    rows_v[t, pl.ds(c * L, L)] = v
                    accs[c % 2] = accs[c % 2] + v
                    accq[c % 2] = accq[c % 2] + v * v
                s_ref[pl.ds(t * L, L)] = accs[0] + accs[1]
                q_ref[pl.ds(t * L, L)] = accq[0] + accq[1]

        make_half(0, sum_v, sq_v)
        make_half(NCH // 2, sum2_v, sq2_v)

        @pl.when(ch < NCHUNK - 1)
        def _():
            @pl.when(ch >= 1)
            def _():
                wait_writeout(ch - 1, 1 - slot)
            gather_in(ch + 1, 1 - slot)
        if b == BATCH - 1:
            @pl.when(j < NCHUNK // BATCH - 1)
            def _():
                pos_in(j + 1)

        @plsc.parallel_loop(0, T)
        def phase_s(t):
            s_tot = sum_v[pl.ds(t * L, L)] + sum2_v[pl.ds(t * L, L)]
            q_tot = sq_v[pl.ds(t * L, L)] + sq2_v[pl.ds(t * L, L)]
            meanv = _lane_sum(s_tot) * (1.0 / HIDDEN)
            varv = _lane_sum(q_tot) * (1.0 / HIDDEN) - meanv * meanv
            rstdv = _rsqrt_vec(varv + EPS)
            p_v[pl.ds(t * L, L)] = rstdv
            q_v[pl.ds(t * L, L)] = meanv * rstdv

        for cg in range(NCG):
            gs = [gam_v[pl.ds((cg * CG + j2) * L, L)] for j2 in range(CG)]
            bs = [bet_v[pl.ds((cg * CG + j2) * L, L)] for j2 in range(CG)]

            @plsc.parallel_loop(0, T)
            def phase_b(t):
                p = p_v[pl.ds(t * L, L)]
                q = q_v[pl.ds(t * L, L)]
                for j2 in range(CG):
                    c = cg * CG + j2
                    v = rows_v[t, pl.ds(c * L, L)]
                    rows_v[t, pl.ds(c * L, L)] = (v * p - q) * gs[j2] + bs[j2]

        writeout(ch, slot)

    def j_body(j, carry):
        for b in range(BATCH):
            compute_chunk(j, b)
        return carry

    lax.fori_loop(0, NCHUNK // BATCH, j_body, 0)
    wait_writeout(NCHUNK - 2, 0)
    wait_writeout(NCHUNK - 1, 1)


def kernel(x, token_table, pos_table, ln_gamma, ln_beta):
    x_flat = x.reshape(-1).astype(jnp.int32)
    out = _embed_ln_kernel(x_flat, token_table, pos_table, ln_gamma, ln_beta)
    return out.reshape(BATCH, SEQ, HIDDEN)


# speedup vs baseline: 2.8031x; 1.0199x over previous
import functools

import jax
import jax.numpy as jnp
from jax import lax
from jax.experimental import pallas as pl
from jax.experimental.pallas import tpu as pltpu
from jax.experimental.pallas import tpu_sc as plsc

VOCAB = 100000
HIDDEN = 768
MAX_POS = 8192
BATCH = 4
SEQ = 4096
EPS = 1e-5

L = 16
NC, NS = 2, 16
NW = NC * NS
NTOK = BATCH * SEQ
TOK_PER_W = NTOK // NW
T = 32
NCHUNK = TOK_PER_W // T
NCH = HIDDEN // L
CG = 8
NCG = NCH // CG

_GDN = lax.GatherDimensionNumbers(
    offset_dims=(), collapsed_slice_dims=(0,), start_index_map=(0,))


def _lane_sum(v):
    for sh in (8, 4, 2, 1):
        idx = (jnp.arange(L, dtype=jnp.int32) + sh) % L
        perm = lax.gather(v, idx[:, None], _GDN, (1,),
                          mode=lax.GatherScatterMode.PROMISE_IN_BOUNDS)
        v = v + perm
    return v


def _rsqrt_vec(v):
    bits = lax.bitcast_convert_type(v, jnp.int32)
    y = lax.bitcast_convert_type(jnp.int32(0x5F3759DF) - (bits >> 1),
                                 jnp.float32)
    for _ in range(3):
        y = y * (1.5 - 0.5 * v * y * y)
    return y


@functools.partial(
    pl.kernel,
    mesh=plsc.VectorSubcoreMesh(core_axis_name="c", subcore_axis_name="s"),
    out_type=jax.ShapeDtypeStruct((NTOK, HIDDEN), jnp.float32),
    scratch_types=[
        pltpu.VMEM((TOK_PER_W,), jnp.int32),
        pltpu.VMEM((4, T, HIDDEN), jnp.float32),
        pltpu.VMEM((T, HIDDEN), jnp.float32),
        pltpu.VMEM((HIDDEN,), jnp.float32),
        pltpu.VMEM((HIDDEN,), jnp.float32),
        pltpu.VMEM((T * L,), jnp.float32),
        pltpu.VMEM((T * L,), jnp.float32),
        pltpu.VMEM((T * L,), jnp.float32),
        pltpu.VMEM((T * L,), jnp.float32),
        pltpu.VMEM((T * L,), jnp.float32),
        pltpu.VMEM((T * L,), jnp.float32),
        pltpu.SemaphoreType.DMA,
        pltpu.SemaphoreType.DMA,
        pltpu.SemaphoreType.DMA,
        pltpu.SemaphoreType.DMA,
        pltpu.SemaphoreType.DMA,
        pltpu.SemaphoreType.DMA,
        pltpu.SemaphoreType.DMA,
        pltpu.SemaphoreType.DMA,
        pltpu.SemaphoreType.DMA,
    ],
)
def _embed_ln_kernel(x_hbm, tok_tbl, pos_tbl, gam_hbm, bet_hbm, out_hbm,
                     idx_v, rows2, pos_v, gam_v, bet_v,
                     sum_v, sq_v, sum2_v, sq2_v, p_v, q_v,
                     semg0, semg1, semg2, semg3, semp,
                     semw0, semw1, semw2, semw3):
    wid = lax.axis_index("s") * NC + lax.axis_index("c")
    w0 = wid * (SEQ // NW)
    pltpu.sync_copy(gam_hbm, gam_v)
    pltpu.sync_copy(bet_hbm, bet_v)
    for b in range(BATCH):
        pltpu.sync_copy(x_hbm.at[pl.ds(b * SEQ + w0, SEQ // NW)],
                        idx_v.at[pl.ds(b * (SEQ // NW), SEQ // NW)])
    semg = (semg0, semg1, semg2, semg3)
    semw = (semw0, semw1, semw2, semw3)

    def idx_off(ch):
        jn = ch // BATCH
        bn = lax.rem(ch, BATCH)
        return bn * (SEQ // NW) + jn * T

    def out_base(ch):
        jn = ch // BATCH
        bn = lax.rem(ch, BATCH)
        return bn * SEQ + w0 + jn * T

    def gather_in(ch, slot):
        pltpu.async_copy(tok_tbl.at[idx_v.at[pl.ds(idx_off(ch), T)]],
                         rows2.at[slot], semg[slot])

    def wait_gather(ch, slot):
        pltpu.make_async_copy(tok_tbl.at[idx_v.at[pl.ds(idx_off(ch), T)]],
                              rows2.at[slot], semg[slot]).wait()

    def pos_in(j):
        pltpu.async_copy(pos_tbl.at[pl.ds(w0 + j * T, T), :], pos_v, semp)

    def wait_pos(j):
        pltpu.make_async_copy(pos_tbl.at[pl.ds(w0 + j * T, T), :],
                              pos_v, semp).wait()

    def writeout(ch, slot):
        pltpu.async_copy(rows2.at[slot],
                         out_hbm.at[pl.ds(out_base(ch), T), :], semw[slot])

    def wait_writeout(ch, slot):
        pltpu.make_async_copy(rows2.at[slot],
                              out_hbm.at[pl.ds(out_base(ch), T), :],
                              semw[slot]).wait()

    gather_in(0, 0)
    gather_in(1, 1)
    pos_in(0)

    def compute_chunk(j, b):
        slot = b
        ch = BATCH * j + b
        rows_v = rows2.at[slot]

        nslot = (b + 2) % BATCH
        @pl.when(ch + 2 <= NCHUNK - 1)
        def _():
            @pl.when(ch >= 2)
            def _():
                wait_writeout(ch - 2, nslot)
            gather_in(ch + 2, nslot)

        wait_gather(ch, slot)
        if b == 0:
            wait_pos(j)

        def make_half(c0, s_ref, q_ref):
            @plsc.parallel_loop(0, T)
            def half(t):
                accs = [jnp.zeros((L,), jnp.float32) for _ in range(2)]
                accq = [jnp.zeros((L,), jnp.float32) for _ in range(2)]
                for c in range(c0, c0 + NCH // 2):
                    v = rows_v[t, pl.ds(c * L, L)] + pos_v[t, pl.ds(c * L, L)]
                    rows_v[t, pl.ds(c * L, L)] = v
                    accs[c % 2] = accs[c % 2] + v
                    accq[c % 2] = accq[c % 2] + v * v
                s_ref[pl.ds(t * L, L)] = accs[0] + accs[1]
                q_ref[pl.ds(t * L, L)] = accq[0] + accq[1]

        make_half(0, sum_v, sq_v)
        make_half(NCH // 2, sum2_v, sq2_v)

        if b == BATCH - 1:
            @pl.when(j < NCHUNK // BATCH - 1)
            def _():
                pos_in(j + 1)

        @plsc.parallel_loop(0, T)
        def phase_s(t):
            s_tot = sum_v[pl.ds(t * L, L)] + sum2_v[pl.ds(t * L, L)]
            q_tot = sq_v[pl.ds(t * L, L)] + sq2_v[pl.ds(t * L, L)]
            meanv = _lane_sum(s_tot) * (1.0 / HIDDEN)
            varv = _lane_sum(q_tot) * (1.0 / HIDDEN) - meanv * meanv
            rstdv = _rsqrt_vec(varv + EPS)
            p_v[pl.ds(t * L, L)] = rstdv
            q_v[pl.ds(t * L, L)] = meanv * rstdv

        for cg in range(NCG):
            gs = [gam_v[pl.ds((cg * CG + j2) * L, L)] for j2 in range(CG)]
            bs = [bet_v[pl.ds((cg * CG + j2) * L, L)] for j2 in range(CG)]

            @plsc.parallel_loop(0, T)
            def phase_b(t):
                p = p_v[pl.ds(t * L, L)]
                q = q_v[pl.ds(t * L, L)]
                for j2 in range(CG):
                    c = cg * CG + j2
                    v = rows_v[t, pl.ds(c * L, L)]
                    rows_v[t, pl.ds(c * L, L)] = (v * p - q) * gs[j2] + bs[j2]

        writeout(ch, slot)

    def j_body(j, carry):
        for b in range(BATCH):
            compute_chunk(j, b)
        return carry

    lax.fori_loop(0, NCHUNK // BATCH, j_body, 0)
    for b in range(BATCH):
        wait_writeout(NCHUNK - BATCH + b, b)


def kernel(x, token_table, pos_table, ln_gamma, ln_beta):
    x_flat = x.reshape(-1).astype(jnp.int32)
    out = _embed_ln_kernel(x_flat, token_table, pos_table, ln_gamma, ln_beta)
    return out.reshape(BATCH, SEQ, HIDDEN)


# speedup vs baseline: 2.8637x; 1.0216x over previous
import functools

import jax
import jax.numpy as jnp
from jax import lax
from jax.experimental import pallas as pl
from jax.experimental.pallas import tpu as pltpu
from jax.experimental.pallas import tpu_sc as plsc

VOCAB = 100000
HIDDEN = 768
MAX_POS = 8192
BATCH = 4
SEQ = 4096
EPS = 1e-5

L = 16
NC, NS = 2, 16
NW = NC * NS
NTOK = BATCH * SEQ
TOK_PER_W = NTOK // NW
T = 32
NCHUNK = TOK_PER_W // T
NCH = HIDDEN // L
CG = 16
NCG = NCH // CG

_GDN = lax.GatherDimensionNumbers(
    offset_dims=(), collapsed_slice_dims=(0,), start_index_map=(0,))


def _lane_sum(v):
    for sh in (8, 4, 2, 1):
        idx = (jnp.arange(L, dtype=jnp.int32) + sh) % L
        perm = lax.gather(v, idx[:, None], _GDN, (1,),
                          mode=lax.GatherScatterMode.PROMISE_IN_BOUNDS)
        v = v + perm
    return v


def _rsqrt_vec(v):
    bits = lax.bitcast_convert_type(v, jnp.int32)
    y = lax.bitcast_convert_type(jnp.int32(0x5F3759DF) - (bits >> 1),
                                 jnp.float32)
    for _ in range(2):
        y = y * (1.5 - 0.5 * v * y * y)
    return y


@functools.partial(
    pl.kernel,
    mesh=plsc.VectorSubcoreMesh(core_axis_name="c", subcore_axis_name="s"),
    out_type=jax.ShapeDtypeStruct((NTOK, HIDDEN), jnp.float32),
    scratch_types=[
        pltpu.VMEM((TOK_PER_W,), jnp.int32),
        pltpu.VMEM((4, T, HIDDEN), jnp.float32),
        pltpu.VMEM((T, HIDDEN), jnp.float32),
        pltpu.VMEM((HIDDEN,), jnp.float32),
        pltpu.VMEM((HIDDEN,), jnp.float32),
        pltpu.VMEM((T * L,), jnp.float32),
        pltpu.VMEM((T * L,), jnp.float32),
        pltpu.VMEM((T * L,), jnp.float32),
        pltpu.VMEM((T * L,), jnp.float32),
        pltpu.VMEM((T * L,), jnp.float32),
        pltpu.VMEM((T * L,), jnp.float32),
        pltpu.SemaphoreType.DMA,
        pltpu.SemaphoreType.DMA,
        pltpu.SemaphoreType.DMA,
        pltpu.SemaphoreType.DMA,
        pltpu.SemaphoreType.DMA,
        pltpu.SemaphoreType.DMA,
        pltpu.SemaphoreType.DMA,
        pltpu.SemaphoreType.DMA,
        pltpu.SemaphoreType.DMA,
    ],
)
def _embed_ln_kernel(x_hbm, tok_tbl, pos_tbl, gam_hbm, bet_hbm, out_hbm,
                     idx_v, rows2, pos_v, gam_v, bet_v,
                     sum_v, sq_v, sum2_v, sq2_v, p_v, q_v,
                     semg0, semg1, semg2, semg3, semp,
                     semw0, semw1, semw2, semw3):
    wid = lax.axis_index("s") * NC + lax.axis_index("c")
    w0 = wid * (SEQ // NW)
    pltpu.sync_copy(gam_hbm, gam_v)
    pltpu.sync_copy(bet_hbm, bet_v)
    for b in range(BATCH):
        pltpu.sync_copy(x_hbm.at[pl.ds(b * SEQ + w0, SEQ // NW)],
                        idx_v.at[pl.ds(b * (SEQ // NW), SEQ // NW)])
    semg = (semg0, semg1, semg2, semg3)
    semw = (semw0, semw1, semw2, semw3)

    def idx_off(ch):
        jn = ch // BATCH
        bn = lax.rem(ch, BATCH)
        return bn * (SEQ // NW) + jn * T

    def out_base(ch):
        jn = ch // BATCH
        bn = lax.rem(ch, BATCH)
        return bn * SEQ + w0 + jn * T

    def gather_in(ch, slot):
        pltpu.async_copy(tok_tbl.at[idx_v.at[pl.ds(idx_off(ch), T)]],
                         rows2.at[slot], semg[slot])

    def wait_gather(ch, slot):
        pltpu.make_async_copy(tok_tbl.at[idx_v.at[pl.ds(idx_off(ch), T)]],
                              rows2.at[slot], semg[slot]).wait()

    def pos_in(j):
        pltpu.async_copy(pos_tbl.at[pl.ds(w0 + j * T, T), :], pos_v, semp)

    def wait_pos(j):
        pltpu.make_async_copy(pos_tbl.at[pl.ds(w0 + j * T, T), :],
                              pos_v, semp).wait()

    def writeout(ch, slot):
        pltpu.async_copy(rows2.at[slot],
                         out_hbm.at[pl.ds(out_base(ch), T), :], semw[slot])

    def wait_writeout(ch, slot):
        pltpu.make_async_copy(rows2.at[slot],
                              out_hbm.at[pl.ds(out_base(ch), T), :],
                              semw[slot]).wait()

    gather_in(0, 0)
    gather_in(1, 1)
    pos_in(0)

    def compute_chunk(j, b):
        slot = b
        ch = BATCH * j + b
        rows_v = rows2.at[slot]

        nslot = (b + 2) % BATCH
        @pl.when(ch + 2 <= NCHUNK - 1)
        def _():
            @pl.when(ch >= 2)
            def _():
                wait_writeout(ch - 2, nslot)
            gather_in(ch + 2, nslot)

        wait_gather(ch, slot)
        if b == 0:
            wait_pos(j)

        def make_half(c0, s_ref, q_ref):
            @plsc.parallel_loop(0, T)
            def half(t):
                accs = [jnp.zeros((L,), jnp.float32) for _ in range(2)]
                accq = [jnp.zeros((L,), jnp.float32) for _ in range(2)]
                for c in range(c0, c0 + NCH // 2):
                    v = rows_v[t, pl.ds(c * L, L)] + pos_v[t, pl.ds(c * L, L)]
                    rows_v[t, pl.ds(c * L, L)] = v
                    accs[c % 2] = accs[c % 2] + v
                    accq[c % 2] = accq[c % 2] + v * v
                s_ref[pl.ds(t * L, L)] = accs[0] + accs[1]
                q_ref[pl.ds(t * L, L)] = accq[0] + accq[1]

        make_half(0, sum_v, sq_v)
        make_half(NCH // 2, sum2_v, sq2_v)

        if b == BATCH - 1:
            @pl.when(j < NCHUNK // BATCH - 1)
            def _():
                pos_in(j + 1)

        @plsc.parallel_loop(0, T)
        def phase_s(t):
            s_tot = sum_v[pl.ds(t * L, L)] + sum2_v[pl.ds(t * L, L)]
            q_tot = sq_v[pl.ds(t * L, L)] + sq2_v[pl.ds(t * L, L)]
            meanv = _lane_sum(s_tot) * (1.0 / HIDDEN)
            varv = _lane_sum(q_tot) * (1.0 / HIDDEN) - meanv * meanv
            rstdv = _rsqrt_vec(varv + EPS)
            p_v[pl.ds(t * L, L)] = rstdv
            q_v[pl.ds(t * L, L)] = meanv * rstdv

        for cg in range(NCG):
            gs = [gam_v[pl.ds((cg * CG + j2) * L, L)] for j2 in range(CG)]
            bs = [bet_v[pl.ds((cg * CG + j2) * L, L)] for j2 in range(CG)]

            @plsc.parallel_loop(0, T)
            def phase_b(t):
                p = p_v[pl.ds(t * L, L)]
                q = q_v[pl.ds(t * L, L)]
                for j2 in range(CG):
                    c = cg * CG + j2
                    v = rows_v[t, pl.ds(c * L, L)]
                    rows_v[t, pl.ds(c * L, L)] = (v * p - q) * gs[j2] + bs[j2]

        writeout(ch, slot)

    def j_body(j, carry):
        for b in range(BATCH):
            compute_chunk(j, b)
        return carry

    lax.fori_loop(0, NCHUNK // BATCH, j_body, 0)
    for b in range(BATCH):
        wait_writeout(NCHUNK - BATCH + b, b)


def kernel(x, token_table, pos_table, ln_gamma, ln_beta):
    x_flat = x.reshape(-1).astype(jnp.int32)
    out = _embed_ln_kernel(x_flat, token_table, pos_table, ln_gamma, ln_beta)
    return out.reshape(BATCH, SEQ, HIDDEN)
